# baseline (speedup 1.0000x reference)
import functools

import jax
import jax.numpy as jnp
from jax import lax
from jax.experimental import pallas as pl
from jax.experimental.pallas import tpu as pltpu
from jax.experimental.pallas import tpu_sc as plsc

B = 16384
D = 32
VU = 1000001
VM = 100001
CW = 512
TW = 640
TWM = 256
NCHU = 1952
NCHM = 195
GRP = 16
NSLOT = 4
SEC = 2048


def _sc_stream_gather(user_id, movie_title, user_table_t, movie_table_t):
    info = plsc.get_sparse_core_info()
    nc = info.num_cores
    nw = nc * info.num_subcores
    mesh = plsc.VectorSubcoreMesh(core_axis_name="c", subcore_axis_name="s")

    @functools.partial(
        pl.kernel,
        mesh=mesh,
        compiler_params=pltpu.CompilerParams(needs_layout_passes=False),
        out_type=(
            jax.ShapeDtypeStruct((B + GRP, 128), jnp.float32),
            jax.ShapeDtypeStruct((B + GRP, 128), jnp.float32),
        ),
        scratch_types=[
            pltpu.VMEM((SEC,), jnp.int32),
            pltpu.VMEM((B + GRP,), jnp.int32),
            pltpu.VMEM((B + GRP,), jnp.int32),
            pltpu.VMEM((B + GRP,), jnp.int32),
            pltpu.VMEM((B + GRP,), jnp.int32),
            pltpu.VMEM((2, D, TW), jnp.float32),
            pltpu.VMEM((NSLOT, GRP, 128), jnp.float32),
            pltpu.VMEM((NSLOT, GRP), jnp.int32),
            pltpu.SemaphoreType.DMA,
            pltpu.SemaphoreType.DMA,
        ],
    )
    def gather_kernel(uid_hbm, mid_hbm, utab_hbm, mtab_hbm, uout_hbm, mout_hbm,
                      idx_v, il_v, pl_v, cl_v, cp_v, chunk_v, rows_v, pos_v,
                      csem, ssem):
        wid = lax.axis_index("s") * nc + lax.axis_index("c")
        lane = lax.iota(jnp.int32, 16)

        def run_table(ids_hbm, tab_hbm, out_hbm, nch, vocab, tail_w):
            c_lo = wid * nch // nw
            c_hi = (wid + 1) * nch // nw
            is_last = wid == nw - 1
            v_lo = c_lo * CW
            v_hi = jnp.where(is_last, vocab, c_hi * CW)

            dump = jnp.full((16,), B + GRP - 1, jnp.int32)

            def sec_body(sidx, offv0):
                pltpu.sync_copy(ids_hbm.at[pl.ds(sidx * SEC, SEC)], idx_v)

                def sel(i, offv):
                    ids = idx_v[pl.ds(i * 16, 16)]
                    m = (ids >= v_lo) & (ids < v_hi)
                    mi = m.astype(jnp.int32)
                    ex = plsc.cumsum(mi) - mi
                    tgt = jnp.where(m, offv + ex, dump)
                    plsc.store_scatter(il_v, [tgt], ids)
                    plsc.store_scatter(
                        pl_v, [tgt], lane + (sidx * SEC + i * 16))
                    return offv + plsc.all_reduce_population_count(m)

                return lax.fori_loop(0, SEC // 16, sel, offv0)

            offv = lax.fori_loop(
                0, B // SEC, sec_body, jnp.zeros((16,), jnp.int32))
            n_w = jnp.sum(offv) // 16
            il_v[pl.ds(n_w, 16)] = jnp.full((16,), -1, jnp.int32)
            pl_v[pl.ds(n_w, 16)] = jnp.full((16,), B, jnp.int32)

            def process(buf, base, hi):
                def resc(i, offv):
                    ids = il_v[pl.ds(i * 16, 16)]
                    pos = pl_v[pl.ds(i * 16, 16)]
                    m = (ids >= base) & (ids < hi)
                    mi = m.astype(jnp.int32)
                    ex = plsc.cumsum(mi) - mi
                    tgt = jnp.where(m, offv + ex, dump)
                    plsc.store_scatter(cl_v, [tgt], ids - base)
                    plsc.store_scatter(cp_v, [tgt], pos)
                    return offv + plsc.all_reduce_population_count(m)

                n_c = jnp.sum(lax.fori_loop(
                    0, (n_w + 15) // 16, resc,
                    jnp.zeros((16,), jnp.int32))) // 16
                cl_v[pl.ds(n_c, 16)] = jnp.full((16,), 0, jnp.int32)
                cp_v[pl.ds(n_c, 16)] = jnp.full((16,), B, jnp.int32)
                n_g = (n_c + 15) // 16

                def grp_body(g, _):
                    slot = g % NSLOT
                    l16 = cl_v[pl.ds(g * 16, 16)]
                    p16 = cp_v[pl.ds(g * 16, 16)]
                    pos_v[slot] = p16
                    for s in range(D):
                        r = plsc.load_gather(
                            chunk_v.at[buf],
                            [jnp.full((16,), s, jnp.int32), l16])
                        plsc.store_scatter(
                            rows_v.at[slot],
                            [lane, jnp.full((16,), s, jnp.int32)], r)
                    pltpu.async_copy(
                        rows_v.at[slot], out_hbm.at[pos_v.at[slot]], ssem)

                    @pl.when(g % NSLOT == NSLOT - 1)
                    def _drain4():
                        def d(j, _2):
                            pltpu.make_async_copy(
                                rows_v.at[0], out_hbm.at[pos_v.at[0]],
                                ssem).wait()
                            return 0
                        lax.fori_loop(0, NSLOT, d, 0)

                    return 0

                lax.fori_loop(0, n_g, grp_body, 0)

                def dfin(j, _):
                    @pl.when(j < n_g % NSLOT)
                    def _w():
                        pltpu.make_async_copy(
                            rows_v.at[0], out_hbm.at[pos_v.at[0]], ssem).wait()
                    return 0

                lax.fori_loop(0, NSLOT, dfin, 0)

            @pl.when(c_lo < c_hi)
            def _prime():
                pltpu.async_copy(
                    tab_hbm.at[:, pl.ds(pl.multiple_of(c_lo * CW, 128), CW)],
                    chunk_v.at[0, :, pl.ds(0, CW)], csem)

            def chunk_body(c, _):
                buf = (c - c_lo) % 2
                pltpu.make_async_copy(
                    tab_hbm.at[:, pl.ds(0, CW)],
                    chunk_v.at[0, :, pl.ds(0, CW)], csem).wait()

                @pl.when(c + 1 < c_hi)
                def _prefetch():
                    pltpu.async_copy(
                        tab_hbm.at[:, pl.ds(
                            pl.multiple_of((c + 1) * CW, 128), CW)],
                        chunk_v.at[1 - buf, :, pl.ds(0, CW)], csem)

                process(buf, c * CW, (c + 1) * CW)
                return 0

            lax.fori_loop(c_lo, c_hi, chunk_body, 0)

            @pl.when(is_last)
            def _tail():
                pltpu.sync_copy(
                    tab_hbm.at[:, pl.ds(pl.multiple_of(nch * CW, 128), tail_w)],
                    chunk_v.at[0, :, pl.ds(0, tail_w)])
                process(0, nch * CW, vocab)

        run_table(uid_hbm, utab_hbm, uout_hbm, NCHU, VU, TW)
        run_table(mid_hbm, mtab_hbm, mout_hbm, NCHM, VM, TWM)

    return gather_kernel(user_id, movie_title, user_table_t, movie_table_t)


def _tc_mlp(ue128, me128, ts, W1, b1, W2, b2, W3, b3):
    BB = 2048

    def body(ue_ref, me_ref, ts_ref, w1u_ref, w1m_ref, w1t_ref, b1_ref,
             w2_ref, b2_ref, w3_ref, b3_ref, out_ref):
        ue = ue_ref[...][:, :D]
        me = me_ref[...][:, :D]
        h = jnp.dot(ue, w1u_ref[...], preferred_element_type=jnp.float32)
        h = h + jnp.dot(me, w1m_ref[...], preferred_element_type=jnp.float32)
        h = h + ts_ref[...] * w1t_ref[...]
        h = jnp.maximum(h + b1_ref[...], 0.0)
        h = jnp.maximum(
            jnp.dot(h, w2_ref[...], preferred_element_type=jnp.float32)
            + b2_ref[...], 0.0)
        out_ref[...] = (
            jnp.dot(h, w3_ref[...], preferred_element_type=jnp.float32)
            + b3_ref[...])

    return pl.pallas_call(
        body,
        grid=(B // BB,),
        in_specs=[
            pl.BlockSpec((BB, 128), lambda i: (i, 0)),
            pl.BlockSpec((BB, 128), lambda i: (i, 0)),
            pl.BlockSpec((BB, 1), lambda i: (i, 0)),
            pl.BlockSpec((D, 256), lambda i: (0, 0)),
            pl.BlockSpec((D, 256), lambda i: (0, 0)),
            pl.BlockSpec((1, 256), lambda i: (0, 0)),
            pl.BlockSpec((1, 256), lambda i: (0, 0)),
            pl.BlockSpec((256, 64), lambda i: (0, 0)),
            pl.BlockSpec((1, 64), lambda i: (0, 0)),
            pl.BlockSpec((64, 1), lambda i: (0, 0)),
            pl.BlockSpec((1, 1), lambda i: (0, 0)),
        ],
        out_specs=pl.BlockSpec((BB, 1), lambda i: (i, 0)),
        out_shape=jax.ShapeDtypeStruct((B, 1), jnp.float32),
    )(ue128, me128, ts, W1[:D], W1[D:2 * D], W1[2 * D:], b1.reshape(1, 256),
      W2, b2.reshape(1, 64), W3, b3.reshape(1, 1))


def kernel(user_id, movie_title, timestamp, user_table, movie_table,
           W1, b1, W2, b2, W3, b3):
    ue128, me128 = _sc_stream_gather(
        user_id.astype(jnp.int32), movie_title.astype(jnp.int32),
        user_table.T, movie_table.T)
    return _tc_mlp(ue128, me128, timestamp.reshape(B, 1),
                   W1, b1, W2, b2, W3, b3)


# speedup vs baseline: 1.0021x; 1.0021x over previous
import functools

import jax
import jax.numpy as jnp
from jax import lax
from jax.experimental import pallas as pl
from jax.experimental.pallas import tpu as pltpu
from jax.experimental.pallas import tpu_sc as plsc

B = 16384
D = 32
VU = 1000001
VM = 100001
CW = 512
TW = 640
TWM = 256
NCHU = 1952
NCHM = 195
GRP = 16
NSLOT = 4
SEC = 2048


def _sc_stream_gather(user_id, movie_title, user_table_t, movie_table_t):
    info = plsc.get_sparse_core_info()
    nc = info.num_cores
    nw = nc * info.num_subcores
    mesh = plsc.VectorSubcoreMesh(core_axis_name="c", subcore_axis_name="s")

    @functools.partial(
        pl.kernel,
        mesh=mesh,
        compiler_params=pltpu.CompilerParams(needs_layout_passes=False),
        out_type=(
            jax.ShapeDtypeStruct((B + GRP, 128), jnp.float32),
            jax.ShapeDtypeStruct((B + GRP, 128), jnp.float32),
        ),
        scratch_types=[
            pltpu.VMEM((SEC,), jnp.int32),
            pltpu.VMEM((B + GRP,), jnp.int32),
            pltpu.VMEM((B + GRP,), jnp.int32),
            pltpu.VMEM((B + GRP,), jnp.int32),
            pltpu.VMEM((B + GRP,), jnp.int32),
            pltpu.VMEM((2, D, TW), jnp.float32),
            pltpu.VMEM((NSLOT, GRP, 128), jnp.float32),
            pltpu.VMEM((NSLOT, GRP), jnp.int32),
            pltpu.SemaphoreType.DMA,
            pltpu.SemaphoreType.DMA,
        ],
    )
    def gather_kernel(uid_hbm, mid_hbm, utab_hbm, mtab_hbm, uout_hbm, mout_hbm,
                      idx_v, il_v, pl_v, cl_v, cp_v, chunk_v, rows_v, pos_v,
                      csem, ssem):
        wid = lax.axis_index("s") * nc + lax.axis_index("c")
        lane = lax.iota(jnp.int32, 16)

        def run_table(ids_hbm, tab_hbm, out_hbm, nch, vocab, tail_w):
            c_lo = wid * nch // nw
            c_hi = (wid + 1) * nch // nw
            is_last = wid == nw - 1
            v_lo = c_lo * CW
            v_hi = jnp.where(is_last, vocab, c_hi * CW)

            dump = jnp.full((16,), B + GRP - 1, jnp.int32)

            def sec_body(sidx, offv0):
                pltpu.sync_copy(ids_hbm.at[pl.ds(sidx * SEC, SEC)], idx_v)

                def sel(i, offv):
                    ids = idx_v[pl.ds(i * 16, 16)]
                    m = (ids >= v_lo) & (ids < v_hi)
                    mi = m.astype(jnp.int32)
                    ex = plsc.cumsum(mi) - mi
                    tgt = jnp.where(m, offv + ex, dump)
                    plsc.store_scatter(il_v, [tgt], ids)
                    plsc.store_scatter(
                        pl_v, [tgt], lane + (sidx * SEC + i * 16))
                    return offv + plsc.all_reduce_population_count(m)

                return lax.fori_loop(0, SEC // 16, sel, offv0)

            with jax.named_scope("sc_sel"):
                offv = lax.fori_loop(
                    0, B // SEC, sec_body, jnp.zeros((16,), jnp.int32))
            n_w = jnp.sum(offv) // 16
            il_v[pl.ds(n_w, 16)] = jnp.full((16,), -1, jnp.int32)
            pl_v[pl.ds(n_w, 16)] = jnp.full((16,), B, jnp.int32)

            def process(buf, base, hi):
                def resc(i, offv):
                    ids = il_v[pl.ds(i * 16, 16)]
                    pos = pl_v[pl.ds(i * 16, 16)]
                    m = (ids >= base) & (ids < hi)
                    mi = m.astype(jnp.int32)
                    ex = plsc.cumsum(mi) - mi
                    tgt = jnp.where(m, offv + ex, dump)
                    plsc.store_scatter(cl_v, [tgt], ids - base)
                    plsc.store_scatter(cp_v, [tgt], pos)
                    return offv + plsc.all_reduce_population_count(m)

                n_c = jnp.sum(lax.fori_loop(
                    0, (n_w + 15) // 16, resc,
                    jnp.zeros((16,), jnp.int32))) // 16
                cl_v[pl.ds(n_c, 16)] = jnp.full((16,), 0, jnp.int32)
                cp_v[pl.ds(n_c, 16)] = jnp.full((16,), B, jnp.int32)
                n_g = (n_c + 15) // 16

                def grp_body(g, _):
                    slot = g % NSLOT
                    l16 = cl_v[pl.ds(g * 16, 16)]
                    p16 = cp_v[pl.ds(g * 16, 16)]
                    pos_v[slot] = p16
                    for s in range(D):
                        r = plsc.load_gather(
                            chunk_v.at[buf],
                            [jnp.full((16,), s, jnp.int32), l16])
                        plsc.store_scatter(
                            rows_v.at[slot],
                            [lane, jnp.full((16,), s, jnp.int32)], r)
                    pltpu.async_copy(
                        rows_v.at[slot], out_hbm.at[pos_v.at[slot]], ssem)

                    @pl.when(g % NSLOT == NSLOT - 1)
                    def _drain4():
                        def d(j, _2):
                            pltpu.make_async_copy(
                                rows_v.at[0], out_hbm.at[pos_v.at[0]],
                                ssem).wait()
                            return 0
                        lax.fori_loop(0, NSLOT, d, 0)

                    return 0

                lax.fori_loop(0, n_g, grp_body, 0)

                def dfin(j, _):
                    @pl.when(j < n_g % NSLOT)
                    def _w():
                        pltpu.make_async_copy(
                            rows_v.at[0], out_hbm.at[pos_v.at[0]], ssem).wait()
                    return 0

                lax.fori_loop(0, NSLOT, dfin, 0)

            @pl.when(c_lo < c_hi)
            def _prime():
                pltpu.async_copy(
                    tab_hbm.at[:, pl.ds(pl.multiple_of(c_lo * CW, 128), CW)],
                    chunk_v.at[0, :, pl.ds(0, CW)], csem)

            def chunk_body(c, _):
                buf = (c - c_lo) % 2
                pltpu.make_async_copy(
                    tab_hbm.at[:, pl.ds(0, CW)],
                    chunk_v.at[0, :, pl.ds(0, CW)], csem).wait()

                @pl.when(c + 1 < c_hi)
                def _prefetch():
                    pltpu.async_copy(
                        tab_hbm.at[:, pl.ds(
                            pl.multiple_of((c + 1) * CW, 128), CW)],
                        chunk_v.at[1 - buf, :, pl.ds(0, CW)], csem)

                process(buf, c * CW, (c + 1) * CW)
                return 0

            with jax.named_scope("sc_stream"):
                lax.fori_loop(c_lo, c_hi, chunk_body, 0)

            @pl.when(is_last)
            def _tail():
                pltpu.sync_copy(
                    tab_hbm.at[:, pl.ds(pl.multiple_of(nch * CW, 128), tail_w)],
                    chunk_v.at[0, :, pl.ds(0, tail_w)])
                process(0, nch * CW, vocab)

        run_table(uid_hbm, utab_hbm, uout_hbm, NCHU, VU, TW)
        run_table(mid_hbm, mtab_hbm, mout_hbm, NCHM, VM, TWM)

    return gather_kernel(user_id, movie_title, user_table_t, movie_table_t)


def _tc_mlp(ue128, me128, ts, W1, b1, W2, b2, W3, b3):
    BB = 2048

    def body(ue_ref, me_ref, ts_ref, w1u_ref, w1m_ref, w1t_ref, b1_ref,
             w2_ref, b2_ref, w3_ref, b3_ref, out_ref):
        ue = ue_ref[...][:, :D]
        me = me_ref[...][:, :D]
        h = jnp.dot(ue, w1u_ref[...], preferred_element_type=jnp.float32)
        h = h + jnp.dot(me, w1m_ref[...], preferred_element_type=jnp.float32)
        h = h + ts_ref[...] * w1t_ref[...]
        h = jnp.maximum(h + b1_ref[...], 0.0)
        h = jnp.maximum(
            jnp.dot(h, w2_ref[...], preferred_element_type=jnp.float32)
            + b2_ref[...], 0.0)
        out_ref[...] = (
            jnp.dot(h, w3_ref[...], preferred_element_type=jnp.float32)
            + b3_ref[...])

    return pl.pallas_call(
        body,
        grid=(B // BB,),
        in_specs=[
            pl.BlockSpec((BB, 128), lambda i: (i, 0)),
            pl.BlockSpec((BB, 128), lambda i: (i, 0)),
            pl.BlockSpec((BB, 1), lambda i: (i, 0)),
            pl.BlockSpec((D, 256), lambda i: (0, 0)),
            pl.BlockSpec((D, 256), lambda i: (0, 0)),
            pl.BlockSpec((1, 256), lambda i: (0, 0)),
            pl.BlockSpec((1, 256), lambda i: (0, 0)),
            pl.BlockSpec((256, 64), lambda i: (0, 0)),
            pl.BlockSpec((1, 64), lambda i: (0, 0)),
            pl.BlockSpec((64, 1), lambda i: (0, 0)),
            pl.BlockSpec((1, 1), lambda i: (0, 0)),
        ],
        out_specs=pl.BlockSpec((BB, 1), lambda i: (i, 0)),
        out_shape=jax.ShapeDtypeStruct((B, 1), jnp.float32),
    )(ue128, me128, ts, W1[:D], W1[D:2 * D], W1[2 * D:], b1.reshape(1, 256),
      W2, b2.reshape(1, 64), W3, b3.reshape(1, 1))


def kernel(user_id, movie_title, timestamp, user_table, movie_table,
           W1, b1, W2, b2, W3, b3):
    ue128, me128 = _sc_stream_gather(
        user_id.astype(jnp.int32), movie_title.astype(jnp.int32),
        user_table.T, movie_table.T)
    return _tc_mlp(ue128, me128, timestamp.reshape(B, 1),
                   W1, b1, W2, b2, W3, b3)


# speedup vs baseline: 1.5990x; 1.5957x over previous
import functools

import jax
import jax.numpy as jnp
from jax import lax
from jax.experimental import pallas as pl
from jax.experimental.pallas import tpu as pltpu
from jax.experimental.pallas import tpu_sc as plsc

B = 16384
D = 32
VU = 1000001
VM = 100001
CW = 1024
TW = 640
TWM = 768
NCHU = 976
NCHM = 97
GRP = 16
NSLOT = 4
SEC = 2048


def _sc_stream_gather(user_id, movie_title, user_table_t, movie_table_t):
    info = plsc.get_sparse_core_info()
    nc = info.num_cores
    nw = nc * info.num_subcores
    mesh = plsc.VectorSubcoreMesh(core_axis_name="c", subcore_axis_name="s")

    @functools.partial(
        pl.kernel,
        mesh=mesh,
        compiler_params=pltpu.CompilerParams(needs_layout_passes=False),
        out_type=(
            jax.ShapeDtypeStruct((B + GRP, 128), jnp.float32),
            jax.ShapeDtypeStruct((B + GRP, 128), jnp.float32),
        ),
        scratch_types=[
            pltpu.VMEM((SEC,), jnp.int32),
            pltpu.VMEM((B + GRP,), jnp.int32),
            pltpu.VMEM((B + GRP,), jnp.int32),
            pltpu.VMEM((2, D, CW), jnp.float32),
            pltpu.VMEM((NSLOT, GRP, 128), jnp.float32),
            pltpu.VMEM((NSLOT, GRP), jnp.int32),
            pltpu.SemaphoreType.DMA,
            pltpu.SemaphoreType.DMA,
        ],
    )
    def gather_kernel(uid_hbm, mid_hbm, utab_hbm, mtab_hbm, uout_hbm, mout_hbm,
                      idx_v, wl_v, cw_v, chunk_v, rows_v, pos_v,
                      csem, ssem):
        wid = lax.axis_index("s") * nc + lax.axis_index("c")
        lane = lax.iota(jnp.int32, 16)

        def run_table(ids_hbm, tab_hbm, out_hbm, nch, vocab, tail_w):
            c_lo = wid * nch // nw
            c_hi = (wid + 1) * nch // nw
            is_last = wid == nw - 1
            v_lo = c_lo * CW
            v_hi = jnp.where(is_last, vocab, c_hi * CW)

            dump = jnp.full((16,), B + GRP - 1, jnp.int32)

            def sec_body(sidx, offv0):
                pltpu.sync_copy(ids_hbm.at[pl.ds(sidx * SEC, SEC)], idx_v)

                def sel(i, offv):
                    ids = idx_v[pl.ds(i * 16, 16)]
                    m = (ids >= v_lo) & (ids < v_hi)
                    mi = m.astype(jnp.int32)
                    ex = plsc.cumsum(mi) - mi
                    tgt = jnp.where(m, offv + ex, dump)
                    packed = (ids - v_lo) | (
                        (lane + (sidx * SEC + i * 16)) << 15)
                    plsc.store_scatter(wl_v, [tgt], packed)
                    return offv + plsc.all_reduce_population_count(m)

                return lax.fori_loop(0, SEC // 16, sel, offv0)

            with jax.named_scope("sc_sel"):
                offv = lax.fori_loop(
                    0, B // SEC, sec_body, jnp.zeros((16,), jnp.int32))
            n_w = jnp.sum(offv) // 16
            wl_v[pl.ds(n_w, 16)] = jnp.full((16,), 0x7FFF | (B << 15),
                                            jnp.int32)

            def process(buf, base, hi):
                wb = base - v_lo

                def resc(i, offv):
                    w = wl_v[pl.ds(i * 16, 16)]
                    rel = w & 0x7FFF
                    m = (rel >= wb) & (rel < wb + (hi - base))
                    mi = m.astype(jnp.int32)
                    ex = plsc.cumsum(mi) - mi
                    tgt = jnp.where(m, offv + ex, dump)
                    plsc.store_scatter(cw_v, [tgt], w - wb)
                    return offv + plsc.all_reduce_population_count(m)

                n_c = jnp.sum(lax.fori_loop(
                    0, (n_w + 15) // 16, resc,
                    jnp.zeros((16,), jnp.int32))) // 16
                cw_v[pl.ds(n_c, 16)] = jnp.full((16,), B << 15, jnp.int32)
                n_g = (n_c + 15) // 16

                def grp_body(g, _):
                    slot = g % NSLOT
                    w16 = cw_v[pl.ds(g * 16, 16)]
                    l16 = w16 & 0x7FFF
                    p16 = w16 >> 15
                    pos_v[slot] = p16
                    for s in range(D):
                        r = plsc.load_gather(
                            chunk_v.at[buf],
                            [jnp.full((16,), s, jnp.int32), l16])
                        plsc.store_scatter(
                            rows_v.at[slot],
                            [lane, jnp.full((16,), s, jnp.int32)], r)
                    pltpu.async_copy(
                        rows_v.at[slot], out_hbm.at[pos_v.at[slot]], ssem)

                    @pl.when(g % NSLOT == NSLOT - 1)
                    def _drain4():
                        def d(j, _2):
                            pltpu.make_async_copy(
                                rows_v.at[0], out_hbm.at[pos_v.at[0]],
                                ssem).wait()
                            return 0
                        lax.fori_loop(0, NSLOT, d, 0)

                    return 0

                lax.fori_loop(0, n_g, grp_body, 0)

                def dfin(j, _):
                    @pl.when(j < n_g % NSLOT)
                    def _w():
                        pltpu.make_async_copy(
                            rows_v.at[0], out_hbm.at[pos_v.at[0]], ssem).wait()
                    return 0

                lax.fori_loop(0, NSLOT, dfin, 0)

            @pl.when(c_lo < c_hi)
            def _prime():
                pltpu.async_copy(
                    tab_hbm.at[:, pl.ds(pl.multiple_of(c_lo * CW, 128), CW)],
                    chunk_v.at[0, :, pl.ds(0, CW)], csem)

            def chunk_body(c, _):
                buf = (c - c_lo) % 2
                pltpu.make_async_copy(
                    tab_hbm.at[:, pl.ds(0, CW)],
                    chunk_v.at[0, :, pl.ds(0, CW)], csem).wait()

                @pl.when(c + 1 < c_hi)
                def _prefetch():
                    pltpu.async_copy(
                        tab_hbm.at[:, pl.ds(
                            pl.multiple_of((c + 1) * CW, 128), CW)],
                        chunk_v.at[1 - buf, :, pl.ds(0, CW)], csem)

                process(buf, c * CW, (c + 1) * CW)
                return 0

            with jax.named_scope("sc_stream"):
                lax.fori_loop(c_lo, c_hi, chunk_body, 0)

            @pl.when(is_last)
            def _tail():
                pltpu.sync_copy(
                    tab_hbm.at[:, pl.ds(pl.multiple_of(nch * CW, 128), tail_w)],
                    chunk_v.at[0, :, pl.ds(0, tail_w)])
                process(0, nch * CW, vocab)

        run_table(uid_hbm, utab_hbm, uout_hbm, NCHU, VU, TW)
        run_table(mid_hbm, mtab_hbm, mout_hbm, NCHM, VM, TWM)

    return gather_kernel(user_id, movie_title, user_table_t, movie_table_t)


def _tc_mlp(ue128, me128, ts, W1, b1, W2, b2, W3, b3):
    BB = 2048

    def body(ue_ref, me_ref, ts_ref, w1u_ref, w1m_ref, w1t_ref, b1_ref,
             w2_ref, b2_ref, w3_ref, b3_ref, out_ref):
        ue = ue_ref[...][:, :D]
        me = me_ref[...][:, :D]
        h = jnp.dot(ue, w1u_ref[...], preferred_element_type=jnp.float32)
        h = h + jnp.dot(me, w1m_ref[...], preferred_element_type=jnp.float32)
        h = h + ts_ref[...] * w1t_ref[...]
        h = jnp.maximum(h + b1_ref[...], 0.0)
        h = jnp.maximum(
            jnp.dot(h, w2_ref[...], preferred_element_type=jnp.float32)
            + b2_ref[...], 0.0)
        out_ref[...] = (
            jnp.dot(h, w3_ref[...], preferred_element_type=jnp.float32)
            + b3_ref[...])

    return pl.pallas_call(
        body,
        grid=(B // BB,),
        in_specs=[
            pl.BlockSpec((BB, 128), lambda i: (i, 0)),
            pl.BlockSpec((BB, 128), lambda i: (i, 0)),
            pl.BlockSpec((BB, 1), lambda i: (i, 0)),
            pl.BlockSpec((D, 256), lambda i: (0, 0)),
            pl.BlockSpec((D, 256), lambda i: (0, 0)),
            pl.BlockSpec((1, 256), lambda i: (0, 0)),
            pl.BlockSpec((1, 256), lambda i: (0, 0)),
            pl.BlockSpec((256, 64), lambda i: (0, 0)),
            pl.BlockSpec((1, 64), lambda i: (0, 0)),
            pl.BlockSpec((64, 1), lambda i: (0, 0)),
            pl.BlockSpec((1, 1), lambda i: (0, 0)),
        ],
        out_specs=pl.BlockSpec((BB, 1), lambda i: (i, 0)),
        out_shape=jax.ShapeDtypeStruct((B, 1), jnp.float32),
    )(ue128, me128, ts, W1[:D], W1[D:2 * D], W1[2 * D:], b1.reshape(1, 256),
      W2, b2.reshape(1, 64), W3, b3.reshape(1, 1))


def kernel(user_id, movie_title, timestamp, user_table, movie_table,
           W1, b1, W2, b2, W3, b3):
    ue128, me128 = _sc_stream_gather(
        user_id.astype(jnp.int32), movie_title.astype(jnp.int32),
        user_table.T, movie_table.T)
    return _tc_mlp(ue128, me128, timestamp.reshape(B, 1),
                   W1, b1, W2, b2, W3, b3)
